# baseline (speedup 1.0000x reference)
import functools

import jax
import jax.numpy as jnp
from jax import lax
from jax.experimental import pallas as pl
from jax.experimental.pallas import tpu as pltpu
from jax.experimental.pallas import tpu_sc as plsc

_B = 16384
_D_PROC = 32
_NC, _NS = 2, 16
_NW = _NC * _NS
_BPW = _B // _NW
_CH = 128
_NCH = _BPW // _CH

_BLK = 2048
_GRID = _B // _BLK


_V = 1000000


def _sc_gather(idx2, table):
    mesh = plsc.VectorSubcoreMesh(core_axis_name="c", subcore_axis_name="s")

    @functools.partial(
        pl.kernel,
        out_type=jax.ShapeDtypeStruct((_B, _D_PROC), jnp.float32),
        mesh=mesh,
        scratch_types=[
            pltpu.VMEM((_NCH, _CH), jnp.int32),
            pltpu.VMEM((_BPW, _D_PROC), jnp.float32),
            pltpu.SemaphoreType.DMA,
        ],
    )
    def k(idx_hbm, table_hbm, out_hbm, idx_v, rows_v, sem):
        wid = lax.axis_index("s") * _NC + lax.axis_index("c")
        pltpu.sync_copy(idx_hbm.at[pl.ds(wid * _NCH, _NCH)], idx_v)
        for j in range(_NCH):
            for v in range(_CH // 16):
                idx16 = idx_v[j, pl.ds(v * 16, 16)]
                for l in range(16):
                    r = idx16[l]
                    pltpu.async_copy(
                        table_hbm.at[pl.ds(r, 1)],
                        rows_v.at[pl.ds(j * _CH + v * 16 + l, 1)],
                        sem,
                    )
        pltpu.make_async_copy(
            table_hbm.at[pl.ds(0, _BPW)], rows_v, sem
        ).wait()
        pltpu.sync_copy(rows_v, out_hbm.at[pl.ds(wid * _BPW, _BPW)])

    return k(idx2, table)


def _partial_body(envT_ref, mT_ref, tt_ref, tw4_ref, tb4_ref,
                  wc_ref, wt_ref, w1b_ref, w1c_ref,
                  bc_ref, bt_ref, b1_ref, hpre_ref):
    f32 = jnp.float32
    dnT = (((0,), (0,)), ((), ()))

    arg4 = tt_ref[...] * tw4_ref[...] + tb4_ref[...]
    row = lax.broadcasted_iota(jnp.int32, arg4.shape, 0)
    t2vT = jnp.where(row == 0, arg4, jnp.sin(arg4))

    envT = envT_ref[...]
    mT = mT_ref[...]
    env_inT = jnp.concatenate([envT * mT, mT], axis=0)

    a_env = jnp.dot(wc_ref[...], w1b_ref[...], preferred_element_type=f32)
    a_t = jnp.dot(wt_ref[...], w1c_ref[...], preferred_element_type=f32)
    bias = (b1_ref[...]
            + jnp.dot(bc_ref[...], w1b_ref[...], preferred_element_type=f32)
            + jnp.dot(bt_ref[...], w1c_ref[...], preferred_element_type=f32))

    hpre_ref[...] = (
        lax.dot_general(env_inT, a_env, dnT, preferred_element_type=f32)
        + lax.dot_general(t2vT, a_t, dnT, preferred_element_type=f32)
        + bias)


def _final_body(pe_ref, hpre_ref, w1a_ref, w2_ref, b2_ref, out_ref):
    f32 = jnp.float32
    h = jnp.maximum(
        hpre_ref[...]
        + jnp.dot(pe_ref[...], w1a_ref[...], preferred_element_type=f32),
        0.0)
    out_ref[...] = jnp.dot(h, w2_ref[...], preferred_element_type=f32) + b2_ref[...]


def _tc_partial(envT, mT, tt, tw4, tb4, wc, wt, w1b, w1c, bc, bt, b1):
    def full_spec(a):
        return pl.BlockSpec(a.shape, lambda i: (0,) * a.ndim)

    return pl.pallas_call(
        _partial_body,
        grid=(_GRID,),
        in_specs=[
            pl.BlockSpec((8, _BLK), lambda i: (0, i)),
            pl.BlockSpec((8, _BLK), lambda i: (0, i)),
            pl.BlockSpec((1, _BLK), lambda i: (0, i)),
            full_spec(tw4), full_spec(tb4), full_spec(wc), full_spec(wt),
            full_spec(w1b), full_spec(w1c),
            full_spec(bc), full_spec(bt), full_spec(b1),
        ],
        out_specs=pl.BlockSpec((_BLK, 128), lambda i: (i, 0)),
        out_shape=jax.ShapeDtypeStruct((_B, 128), jnp.float32),
        compiler_params=pltpu.CompilerParams(
            dimension_semantics=("parallel",),
        ),
    )(envT, mT, tt, tw4, tb4, wc, wt, w1b, w1c, bc, bt, b1)


def _tc_final(pe, hpre, w1a, w2, b2):
    def full_spec(a):
        return pl.BlockSpec(a.shape, lambda i: (0,) * a.ndim)

    return pl.pallas_call(
        _final_body,
        grid=(_GRID,),
        in_specs=[
            pl.BlockSpec((_BLK, _D_PROC), lambda i: (i, 0)),
            pl.BlockSpec((_BLK, 128), lambda i: (i, 0)),
            full_spec(w1a), full_spec(w2), full_spec(b2),
        ],
        out_specs=pl.BlockSpec((_BLK, 64), lambda i: (i, 0)),
        out_shape=jax.ShapeDtypeStruct((_B, 64), jnp.float32),
        compiler_params=pltpu.CompilerParams(
            dimension_semantics=("parallel",),
        ),
    )(pe, hpre, w1a, w2, b2)


def kernel(process_id, env_cont, env_cont_mask, timestamp, proc_table,
           Wc, bc, t2v_lw, t2v_lb, t2v_pw, t2v_pb, Wt, bt, W1, b1, W2, b2):
    idx2 = process_id.astype(jnp.int32).reshape(_B // _CH, _CH)
    pe = _sc_gather(idx2, proc_table)

    envT = env_cont.T
    mT = env_cont_mask.astype(jnp.float32).T
    tt = timestamp.reshape(1, _B)
    tw4 = jnp.concatenate([t2v_lw, t2v_pw]).reshape(4, 1)
    tb4 = jnp.concatenate([t2v_lb, t2v_pb]).reshape(4, 1)
    w1a, w1b, w1c = W1[:_D_PROC], W1[_D_PROC:_D_PROC + 32], W1[_D_PROC + 32:]
    hpre = _tc_partial(envT, mT, tt, tw4, tb4, Wc, Wt, w1b, w1c,
                       bc.reshape(1, -1), bt.reshape(1, -1), b1.reshape(1, -1))
    return _tc_final(pe, hpre, w1a, W2, b2.reshape(1, -1))


# speedup vs baseline: 1.0124x; 1.0124x over previous
import functools

import jax
import jax.numpy as jnp
from jax import lax
from jax.experimental import pallas as pl
from jax.experimental.pallas import tpu as pltpu
from jax.experimental.pallas import tpu_sc as plsc

_B = 16384
_D_PROC = 32
_NC, _NS = 2, 16
_NW = _NC * _NS
_BPW = _B // _NW
_CH = 128
_NCH = _BPW // _CH

_BLK = 2048
_GRID = _B // _BLK


_V = 1000000


def _sc_gather(idx2, table):
    mesh = plsc.VectorSubcoreMesh(core_axis_name="c", subcore_axis_name="s")

    @functools.partial(
        pl.kernel,
        out_type=jax.ShapeDtypeStruct((_B, _D_PROC), jnp.float32),
        mesh=mesh,
        scratch_types=[
            pltpu.VMEM((_NCH, _CH), jnp.int32),
            pltpu.VMEM((_BPW, _D_PROC), jnp.float32),
            pltpu.SemaphoreType.DMA,
        ],
    )
    def k(idx_hbm, table_hbm, out_hbm, idx_v, rows_v, sem):
        wid = lax.axis_index("s") * _NC + lax.axis_index("c")
        pltpu.sync_copy(idx_hbm.at[pl.ds(wid * _NCH, _NCH)], idx_v)
        for j in range(_NCH):
            for v in range(_CH // 16):
                idx16 = idx_v[j, pl.ds(v * 16, 16)]
                for l in range(16):
                    r = idx16[l]
                    pltpu.async_copy(
                        table_hbm.at[pl.ds(r, 1)],
                        rows_v.at[pl.ds(j * _CH + v * 16 + l, 1)],
                        sem,
                    )
        pltpu.make_async_copy(
            table_hbm.at[pl.ds(0, _BPW)], rows_v, sem
        ).wait()
        pltpu.sync_copy(rows_v, out_hbm.at[pl.ds(wid * _BPW, _BPW)])

    return k(idx2, table)


def _dense_body(pe_ref, envT_ref, mT_ref, tt_ref, tw4_ref, tb4_ref,
                wc_ref, wt_ref, w1a_ref, w1b_ref, w1c_ref,
                bc_ref, bt_ref, b1_ref, w2_ref, b2_ref, out_ref):
    f32 = jnp.float32
    dnT = (((0,), (0,)), ((), ()))

    arg4 = tt_ref[...] * tw4_ref[...] + tb4_ref[...]
    row = lax.broadcasted_iota(jnp.int32, arg4.shape, 0)
    t2vT = jnp.where(row == 0, arg4, jnp.sin(arg4))

    envT = envT_ref[...]
    mT = mT_ref[...]
    env_inT = jnp.concatenate([envT * mT, mT], axis=0)

    a_env = jnp.dot(wc_ref[...], w1b_ref[...], preferred_element_type=f32)
    a_t = jnp.dot(wt_ref[...], w1c_ref[...], preferred_element_type=f32)
    bias = (b1_ref[...]
            + jnp.dot(bc_ref[...], w1b_ref[...], preferred_element_type=f32)
            + jnp.dot(bt_ref[...], w1c_ref[...], preferred_element_type=f32))

    h = (jnp.dot(pe_ref[...], w1a_ref[...], preferred_element_type=f32)
         + lax.dot_general(env_inT, a_env, dnT, preferred_element_type=f32)
         + lax.dot_general(t2vT, a_t, dnT, preferred_element_type=f32)
         + bias)
    h = jnp.maximum(h, 0.0)
    out_ref[...] = jnp.dot(h, w2_ref[...], preferred_element_type=f32) + b2_ref[...]


def _tc_dense(pe, envT, mT, tt, tw4, tb4, wc, wt, w1a, w1b, w1c,
              bc, bt, b1, w2, b2):
    def full_spec(a):
        return pl.BlockSpec(a.shape, lambda i: (0,) * a.ndim)

    return pl.pallas_call(
        _dense_body,
        grid=(_GRID,),
        in_specs=[
            pl.BlockSpec((_BLK, _D_PROC), lambda i: (i, 0)),
            pl.BlockSpec((8, _BLK), lambda i: (0, i)),
            pl.BlockSpec((8, _BLK), lambda i: (0, i)),
            pl.BlockSpec((1, _BLK), lambda i: (0, i)),
            full_spec(tw4), full_spec(tb4), full_spec(wc), full_spec(wt),
            full_spec(w1a), full_spec(w1b), full_spec(w1c),
            full_spec(bc), full_spec(bt), full_spec(b1), full_spec(w2),
            full_spec(b2),
        ],
        out_specs=pl.BlockSpec((_BLK, 64), lambda i: (i, 0)),
        out_shape=jax.ShapeDtypeStruct((_B, 64), jnp.float32),
        compiler_params=pltpu.CompilerParams(
            dimension_semantics=("parallel",),
        ),
    )(pe, envT, mT, tt, tw4, tb4, wc, wt, w1a, w1b, w1c, bc, bt, b1, w2, b2)


def kernel(process_id, env_cont, env_cont_mask, timestamp, proc_table,
           Wc, bc, t2v_lw, t2v_lb, t2v_pw, t2v_pb, Wt, bt, W1, b1, W2, b2):
    idx2 = process_id.astype(jnp.int32).reshape(_B // _CH, _CH)
    pe = _sc_gather(idx2, proc_table)

    envT = env_cont.T
    mT = env_cont_mask.astype(jnp.float32).T
    tt = timestamp.reshape(1, _B)
    tw4 = jnp.concatenate([t2v_lw, t2v_pw]).reshape(4, 1)
    tb4 = jnp.concatenate([t2v_lb, t2v_pb]).reshape(4, 1)
    w1a, w1b, w1c = W1[:_D_PROC], W1[_D_PROC:_D_PROC + 32], W1[_D_PROC + 32:]
    return _tc_dense(pe, envT, mT, tt, tw4, tb4, Wc, Wt, w1a, w1b, w1c,
                     bc.reshape(1, -1), bt.reshape(1, -1), b1.reshape(1, -1),
                     W2, b2.reshape(1, -1))
